# baseline (speedup 1.0000x reference)
import functools

import jax
import jax.numpy as jnp
from jax import lax
from jax.experimental import pallas as pl
from jax.experimental.pallas import tpu as pltpu
from jax.experimental.pallas import tpu_sc as plsc

PRE_SEQ_LEN = 64
HIDDEN = 1024
OUT_DIM = 2 * 24 * 1024
BATCH = 8
ROWS = BATCH * PRE_SEQ_LEN
BN = 4096

NCH = 8
CH = OUT_DIM // NCH
NUM_TILES = 32
ROWS_PER_TILE = ROWS // NUM_TILES


def _mlp_body(table_ref, W1_ref, b1_ref, W2_ref, b2_ref, out_ref, h_ref):
    j = pl.program_id(0)

    @pl.when(j == 0)
    def _():
        emb = table_ref[...]
        h_ref[...] = jnp.tanh(
            jnp.dot(emb, W1_ref[...], preferred_element_type=jnp.float32)
            + b1_ref[...])

    out_ref[...] = (
        jnp.dot(h_ref[...], W2_ref[...], preferred_element_type=jnp.float32)
        + b2_ref[...])


def _mlp_out_all(table, W1, b1r, W2, b2r):
    return pl.pallas_call(
        _mlp_body,
        grid=(OUT_DIM // BN,),
        in_specs=[
            pl.BlockSpec((PRE_SEQ_LEN, HIDDEN), lambda j: (0, 0)),
            pl.BlockSpec((HIDDEN, HIDDEN), lambda j: (0, 0)),
            pl.BlockSpec((1, HIDDEN), lambda j: (0, 0)),
            pl.BlockSpec((HIDDEN, BN), lambda j: (0, j)),
            pl.BlockSpec((1, BN), lambda j: (0, j)),
        ],
        out_specs=pl.BlockSpec((PRE_SEQ_LEN, BN), lambda j: (0, j)),
        out_shape=jax.ShapeDtypeStruct((PRE_SEQ_LEN, OUT_DIM), jnp.float32),
        scratch_shapes=[pltpu.VMEM((PRE_SEQ_LEN, HIDDEN), jnp.float32)],
    )(table, W1, b1r, W2, b2r)


def _make_sc_expand():
    mesh = plsc.VectorSubcoreMesh(core_axis_name="c", subcore_axis_name="s")
    info = plsc.get_sparse_core_info()
    nc = info.num_cores

    @functools.partial(
        pl.kernel,
        mesh=mesh,
        out_type=jax.ShapeDtypeStruct((ROWS * NCH, CH), jnp.float32),
        scratch_types=[
            pltpu.VMEM((ROWS_PER_TILE, NCH), jnp.int32),
            pltpu.VMEM((2, NCH, CH), jnp.float32),
            pltpu.SemaphoreType.DMA,
            pltpu.SemaphoreType.DMA,
        ],
    )
    def sc_expand(src_hbm, idx_hbm, out_hbm, idx_v, buf, g_sem, w_sem):
        wid = lax.axis_index("s") * nc + lax.axis_index("c")
        base = wid * ROWS_PER_TILE
        pltpu.sync_copy(idx_hbm.at[pl.ds(base, ROWS_PER_TILE)], idx_v)

        def do_row(i, slot):
            pltpu.async_copy(src_hbm.at[idx_v.at[i]], buf.at[slot],
                             g_sem).wait()
            pltpu.async_copy(
                buf.at[slot],
                out_hbm.at[pl.ds((base + i) * NCH, NCH)], w_sem).wait()

        def body(i, carry):
            for s in range(2):
                do_row(i + s, s)
            return carry

        lax.fori_loop(0, ROWS_PER_TILE // 2, lambda k, c: body(k * 2, c), 0,
                      unroll=False)

    return sc_expand


_SC_EXPAND = _make_sc_expand()


def kernel(prefix, table, W1, b1, W2, b2):
    b1r = b1.reshape(1, HIDDEN)
    b2r = b2.reshape(1, OUT_DIM)
    out_all = _mlp_out_all(table, W1, b1r, W2, b2r)
    src = out_all.reshape(PRE_SEQ_LEN * NCH, CH)
    pf = prefix.reshape(ROWS, 1).astype(jnp.int32)
    idx = pf * NCH + jnp.arange(NCH, dtype=jnp.int32)[None, :]
    out = _SC_EXPAND(src, idx)
    return out.reshape(BATCH, PRE_SEQ_LEN, OUT_DIM)


# speedup vs baseline: 1.0418x; 1.0418x over previous
import functools

import jax
import jax.numpy as jnp
from jax import lax
from jax.experimental import pallas as pl
from jax.experimental.pallas import tpu as pltpu
from jax.experimental.pallas import tpu_sc as plsc

PRE_SEQ_LEN = 64
HIDDEN = 1024
OUT_DIM = 2 * 24 * 1024
BATCH = 8
ROWS = BATCH * PRE_SEQ_LEN
BN = 4096

NCH = 8
CH = OUT_DIM // NCH
NUM_TILES = 32
ROWS_PER_TILE = ROWS // NUM_TILES
USUB = 4
NB = 4
UNITS = ROWS_PER_TILE * NCH // USUB
FB = UNITS // NB


def _mlp_body(table_ref, W1_ref, b1_ref, W2_ref, b2_ref, out_ref, h_ref):
    j = pl.program_id(0)

    @pl.when(j == 0)
    def _():
        emb = table_ref[...]
        h_ref[...] = jnp.tanh(
            jnp.dot(emb, W1_ref[...], preferred_element_type=jnp.float32)
            + b1_ref[...])

    out_ref[...] = (
        jnp.dot(h_ref[...], W2_ref[...], preferred_element_type=jnp.float32)
        + b2_ref[...])


def _mlp_out_all(table, W1, b1r, W2, b2r):
    return pl.pallas_call(
        _mlp_body,
        grid=(OUT_DIM // BN,),
        in_specs=[
            pl.BlockSpec((PRE_SEQ_LEN, HIDDEN), lambda j: (0, 0)),
            pl.BlockSpec((HIDDEN, HIDDEN), lambda j: (0, 0)),
            pl.BlockSpec((1, HIDDEN), lambda j: (0, 0)),
            pl.BlockSpec((HIDDEN, BN), lambda j: (0, j)),
            pl.BlockSpec((1, BN), lambda j: (0, j)),
        ],
        out_specs=pl.BlockSpec((PRE_SEQ_LEN, BN), lambda j: (0, j)),
        out_shape=jax.ShapeDtypeStruct((PRE_SEQ_LEN, OUT_DIM), jnp.float32),
        scratch_shapes=[pltpu.VMEM((PRE_SEQ_LEN, HIDDEN), jnp.float32)],
    )(table, W1, b1r, W2, b2r)


def _make_sc_expand():
    mesh = plsc.VectorSubcoreMesh(core_axis_name="c", subcore_axis_name="s")
    info = plsc.get_sparse_core_info()
    nc = info.num_cores

    @functools.partial(
        pl.kernel,
        mesh=mesh,
        out_type=jax.ShapeDtypeStruct((ROWS * NCH, CH), jnp.float32),
        scratch_types=[
            pltpu.VMEM((UNITS, USUB), jnp.int32),
            pltpu.VMEM((NB, USUB, CH), jnp.float32),
            pltpu.SemaphoreType.DMA,
            pltpu.SemaphoreType.DMA,
        ],
    )
    def sc_expand(src_hbm, idx_hbm, out_hbm, idx_v, buf, g_sem, w_sem):
        wid = lax.axis_index("s") * nc + lax.axis_index("c")
        ubase = wid * UNITS
        pltpu.sync_copy(idx_hbm.at[pl.ds(ubase, UNITS)], idx_v)

        def g_start(u, s):
            pltpu.async_copy(src_hbm.at[idx_v.at[u]], buf.at[s], g_sem)

        def g_wait(s):
            pltpu.make_async_copy(src_hbm.at[pl.ds(0, USUB)], buf.at[s],
                                  g_sem).wait()

        def w_start(u, s):
            pltpu.async_copy(buf.at[s],
                             out_hbm.at[pl.ds((ubase + u) * USUB, USUB)],
                             w_sem)

        def w_wait(s):
            pltpu.make_async_copy(src_hbm.at[pl.ds(0, USUB)], buf.at[s],
                                  w_sem).wait()

        for b in range(NB):
            g_start(b, b)

        def block(k, carry):
            for b in range(NB):
                u = k * NB + b
                g_wait(b)
                w_start(u, b)
                w_wait(b)
                g_start(u + NB, b)
            return carry

        lax.fori_loop(0, FB - 1, block, 0, unroll=False)
        for b in range(NB):
            u = (FB - 1) * NB + b
            g_wait(b)
            w_start(u, b)
        for b in range(NB):
            w_wait(b)

    return sc_expand


_SC_EXPAND = _make_sc_expand()


def kernel(prefix, table, W1, b1, W2, b2):
    b1r = b1.reshape(1, HIDDEN)
    b2r = b2.reshape(1, OUT_DIM)
    out_all = _mlp_out_all(table, W1, b1r, W2, b2r)
    src = out_all.reshape(PRE_SEQ_LEN * NCH, CH)
    pf = prefix.reshape(ROWS, 1).astype(jnp.int32)
    idx = (pf * NCH + jnp.arange(NCH, dtype=jnp.int32)[None, :]).reshape(
        ROWS * NCH // USUB, USUB)
    out = _SC_EXPAND(src, idx)
    return out.reshape(BATCH, PRE_SEQ_LEN, OUT_DIM)


# speedup vs baseline: 1.7508x; 1.6805x over previous
import functools

import jax
import jax.numpy as jnp
from jax import lax
from jax.experimental import pallas as pl
from jax.experimental.pallas import tpu as pltpu
from jax.experimental.pallas import tpu_sc as plsc

PRE_SEQ_LEN = 64
HIDDEN = 1024
OUT_DIM = 2 * 24 * 1024
BATCH = 8
ROWS = BATCH * PRE_SEQ_LEN
BN = 4096

NCH = 1
CH = OUT_DIM // NCH
NUM_TILES = 32
ROWS_PER_TILE = ROWS // NUM_TILES
USUB = 1
NB = 2
UNITS = ROWS_PER_TILE * NCH // USUB
FB = UNITS // NB


def _mlp_body(table_ref, W1_ref, b1_ref, W2_ref, b2_ref, out_ref, h_ref):
    j = pl.program_id(0)

    @pl.when(j == 0)
    def _():
        emb = table_ref[...]
        h_ref[...] = jnp.tanh(
            jnp.dot(emb, W1_ref[...], preferred_element_type=jnp.float32)
            + b1_ref[...])

    out_ref[...] = (
        jnp.dot(h_ref[...], W2_ref[...], preferred_element_type=jnp.float32)
        + b2_ref[...])


def _mlp_out_all(table, W1, b1r, W2, b2r):
    return pl.pallas_call(
        _mlp_body,
        grid=(OUT_DIM // BN,),
        in_specs=[
            pl.BlockSpec((PRE_SEQ_LEN, HIDDEN), lambda j: (0, 0)),
            pl.BlockSpec((HIDDEN, HIDDEN), lambda j: (0, 0)),
            pl.BlockSpec((1, HIDDEN), lambda j: (0, 0)),
            pl.BlockSpec((HIDDEN, BN), lambda j: (0, j)),
            pl.BlockSpec((1, BN), lambda j: (0, j)),
        ],
        out_specs=pl.BlockSpec((PRE_SEQ_LEN, BN), lambda j: (0, j)),
        out_shape=jax.ShapeDtypeStruct((PRE_SEQ_LEN, OUT_DIM), jnp.float32),
        scratch_shapes=[pltpu.VMEM((PRE_SEQ_LEN, HIDDEN), jnp.float32)],
    )(table, W1, b1r, W2, b2r)


def _make_sc_expand():
    mesh = plsc.VectorSubcoreMesh(core_axis_name="c", subcore_axis_name="s")
    info = plsc.get_sparse_core_info()
    nc = info.num_cores

    @functools.partial(
        pl.kernel,
        mesh=mesh,
        out_type=jax.ShapeDtypeStruct((ROWS * NCH, CH), jnp.float32),
        scratch_types=[
            pltpu.VMEM((UNITS, USUB), jnp.int32),
            pltpu.VMEM((NB, USUB, CH), jnp.float32),
            pltpu.SemaphoreType.DMA,
            pltpu.SemaphoreType.DMA,
        ],
    )
    def sc_expand(src_hbm, idx_hbm, out_hbm, idx_v, buf, g_sem, w_sem):
        wid = lax.axis_index("s") * nc + lax.axis_index("c")
        ubase = wid * UNITS
        pltpu.sync_copy(idx_hbm.at[pl.ds(ubase, UNITS)], idx_v)

        def g_start(u, s):
            pltpu.async_copy(src_hbm.at[idx_v.at[u]], buf.at[s], g_sem)

        def g_wait(s):
            pltpu.make_async_copy(src_hbm.at[pl.ds(0, USUB)], buf.at[s],
                                  g_sem).wait()

        def w_start(u, s):
            pltpu.async_copy(buf.at[s],
                             out_hbm.at[pl.ds((ubase + u) * USUB, USUB)],
                             w_sem)

        def w_wait(s):
            pltpu.make_async_copy(src_hbm.at[pl.ds(0, USUB)], buf.at[s],
                                  w_sem).wait()

        for b in range(NB):
            g_start(b, b)

        def block(k, carry):
            for b in range(NB):
                u = k * NB + b
                g_wait(b)
                w_start(u, b)
                w_wait(b)
                g_start(u + NB, b)
            return carry

        lax.fori_loop(0, FB - 1, block, 0, unroll=False)
        for b in range(NB):
            u = (FB - 1) * NB + b
            g_wait(b)
            w_start(u, b)
        for b in range(NB):
            w_wait(b)

    return sc_expand


_SC_EXPAND = _make_sc_expand()


def kernel(prefix, table, W1, b1, W2, b2):
    b1r = b1.reshape(1, HIDDEN)
    b2r = b2.reshape(1, OUT_DIM)
    out_all = _mlp_out_all(table, W1, b1r, W2, b2r)
    idx = prefix.reshape(ROWS, USUB).astype(jnp.int32)
    out = _SC_EXPAND(out_all, idx)
    return out.reshape(BATCH, PRE_SEQ_LEN, OUT_DIM)


# speedup vs baseline: 2.7509x; 1.5712x over previous
import jax
import jax.numpy as jnp
from jax.experimental import pallas as pl
from jax.experimental.pallas import tpu as pltpu

PRE_SEQ_LEN = 64
HIDDEN = 1024
OUT_DIM = 2 * 24 * 1024
BATCH = 8
ROWS = BATCH * PRE_SEQ_LEN
BN = 4096


def _body(pf_ref, table_ref, W1_ref, b1_ref, W2_ref, b2_ref, out_ref,
          h_ref, p_ref):
    j = pl.program_id(0)

    @pl.when(j == 0)
    def _():
        emb = table_ref[...]
        h = jnp.tanh(
            jnp.dot(emb, W1_ref[...], preferred_element_type=jnp.float32)
            + b1_ref[...])
        h_ref[...] = h
        pf = pf_ref[...]
        iota = jax.lax.broadcasted_iota(jnp.int32, (ROWS, PRE_SEQ_LEN), 1)
        p_ref[...] = (pf == iota).astype(jnp.float32)

    ob = (jnp.dot(h_ref[...], W2_ref[...], preferred_element_type=jnp.float32)
          + b2_ref[...])
    out_ref[...] = jnp.dot(p_ref[...], ob, preferred_element_type=jnp.float32)


def kernel(prefix, table, W1, b1, W2, b2):
    pf2d = prefix.reshape(ROWS, 1).astype(jnp.int32)
    b1r = b1.reshape(1, HIDDEN)
    b2r = b2.reshape(1, OUT_DIM)
    grid = (OUT_DIM // BN,)
    out = pl.pallas_call(
        _body,
        grid=grid,
        in_specs=[
            pl.BlockSpec((ROWS, 1), lambda j: (0, 0)),
            pl.BlockSpec((PRE_SEQ_LEN, HIDDEN), lambda j: (0, 0)),
            pl.BlockSpec((HIDDEN, HIDDEN), lambda j: (0, 0)),
            pl.BlockSpec((1, HIDDEN), lambda j: (0, 0)),
            pl.BlockSpec((HIDDEN, BN), lambda j: (0, j)),
            pl.BlockSpec((1, BN), lambda j: (0, j)),
        ],
        out_specs=pl.BlockSpec((ROWS, BN), lambda j: (0, j)),
        out_shape=jax.ShapeDtypeStruct((ROWS, OUT_DIM), jnp.float32),
        scratch_shapes=[
            pltpu.VMEM((PRE_SEQ_LEN, HIDDEN), jnp.float32),
            pltpu.VMEM((ROWS, PRE_SEQ_LEN), jnp.float32),
        ],
    )(pf2d, table, W1, b1r, W2, b2r)
    return out.reshape(BATCH, PRE_SEQ_LEN, OUT_DIM)
